# Initial kernel scaffold; baseline (speedup 1.0000x reference)
import jax
import jax.numpy as jnp
from jax.experimental import pallas as pl


def kernel(inputs, table):
    raise NotImplementedError("write your pallas kernel here")


# speedup vs baseline: 183.0229x; 183.0229x over previous
import functools

import jax
import jax.numpy as jnp
from jax import lax
from jax.experimental import pallas as pl
from jax.experimental.pallas import tpu as pltpu, tpu_sc as plsc

_NC = 2
_NS = 16
_NW = _NC * _NS
_L = 16

_TOTAL = 4096 * 200
_CHUNK = _TOTAL // _NW
_VECS = _CHUNK // _L
_TABLE = 256


def _build_sc_call():
    mesh = plsc.VectorSubcoreMesh(core_axis_name="c", subcore_axis_name="s")

    @functools.partial(
        pl.kernel,
        out_type=jax.ShapeDtypeStruct((_TOTAL,), jnp.int32),
        mesh=mesh,
        scratch_types=[
            pltpu.VMEM((_TABLE,), jnp.int32),
            pltpu.VMEM((_CHUNK,), jnp.int32),
            pltpu.VMEM((_CHUNK,), jnp.int32),
        ],
        compiler_params=pltpu.CompilerParams(needs_layout_passes=False),
    )
    def lookup(inp_hbm, tab_hbm, out_hbm, tab_v, inp_v, out_v):
        wid = lax.axis_index("s") * _NC + lax.axis_index("c")
        base = wid * _CHUNK
        pltpu.sync_copy(tab_hbm, tab_v)
        pltpu.sync_copy(inp_hbm.at[pl.ds(base, _CHUNK)], inp_v)

        @pl.loop(0, _VECS, unroll=8)
        def _(i):
            off = i * _L
            idx = inp_v[pl.ds(off, _L)]
            out_v[pl.ds(off, _L)] = plsc.load_gather(tab_v, [idx])

        pltpu.sync_copy(out_v, out_hbm.at[pl.ds(base, _CHUNK)])

    return lookup


_lookup = _build_sc_call()


@jax.jit
def kernel(inputs, table):
    flat = inputs.reshape(_TOTAL)
    out = _lookup(flat, table)
    return out.reshape(inputs.shape)


# speedup vs baseline: 230.9117x; 1.2617x over previous
import functools

import jax
import jax.numpy as jnp
from jax import lax
from jax.experimental import pallas as pl
from jax.experimental.pallas import tpu as pltpu, tpu_sc as plsc

_NC = 2
_NS = 16
_NW = _NC * _NS
_L = 16

_TOTAL = 4096 * 200
_CHUNK = _TOTAL // _NW
_VECS = _CHUNK // _L
_TABLE = 256


def _build_sc_call():
    mesh = plsc.VectorSubcoreMesh(core_axis_name="c", subcore_axis_name="s")

    @functools.partial(
        pl.kernel,
        out_type=jax.ShapeDtypeStruct((_TOTAL,), jnp.int32),
        mesh=mesh,
        scratch_types=[
            pltpu.VMEM((_TABLE,), jnp.int32),
            pltpu.VMEM((_CHUNK,), jnp.int32),
            pltpu.VMEM((_CHUNK,), jnp.int32),
        ],
        compiler_params=pltpu.CompilerParams(needs_layout_passes=False),
    )
    def lookup(inp_hbm, tab_hbm, out_hbm, tab_v, inp_v, out_v):
        wid = lax.axis_index("s") * _NC + lax.axis_index("c")
        base = wid * _CHUNK
        pltpu.sync_copy(tab_hbm, tab_v)
        pltpu.sync_copy(inp_hbm.at[pl.ds(base, _CHUNK)], inp_v)

        @plsc.parallel_loop(0, _VECS, unroll=8)
        def _(i):
            off = i * _L
            idx = inp_v[pl.ds(off, _L)]
            out_v[pl.ds(off, _L)] = plsc.load_gather(tab_v, [idx])

        pltpu.sync_copy(out_v, out_hbm.at[pl.ds(base, _CHUNK)])

    return lookup


_lookup = _build_sc_call()


@jax.jit
def kernel(inputs, table):
    flat = inputs.reshape(_TOTAL)
    out = _lookup(flat, table)
    return out.reshape(inputs.shape)


# speedup vs baseline: 232.1507x; 1.0054x over previous
import functools

import jax
import jax.numpy as jnp
from jax import lax
from jax.experimental import pallas as pl
from jax.experimental.pallas import tpu as pltpu, tpu_sc as plsc

_NC = 2
_NS = 16
_NW = _NC * _NS
_L = 16

_TOTAL = 4096 * 200
_CHUNK = _TOTAL // _NW
_VECS = _CHUNK // _L
_TABLE = 256


def _build_sc_call():
    mesh = plsc.VectorSubcoreMesh(core_axis_name="c", subcore_axis_name="s")

    @functools.partial(
        pl.kernel,
        out_type=jax.ShapeDtypeStruct((_TOTAL,), jnp.int32),
        mesh=mesh,
        scratch_types=[
            pltpu.VMEM((_TABLE,), jnp.int32),
            pltpu.VMEM((_CHUNK,), jnp.int32),
            pltpu.VMEM((_CHUNK,), jnp.int32),
        ],
        compiler_params=pltpu.CompilerParams(needs_layout_passes=False),
    )
    def lookup(inp_hbm, tab_hbm, out_hbm, tab_v, inp_v, out_v):
        wid = lax.axis_index("s") * _NC + lax.axis_index("c")
        base = wid * _CHUNK
        pltpu.sync_copy(tab_hbm, tab_v)
        pltpu.sync_copy(inp_hbm.at[pl.ds(base, _CHUNK)], inp_v)

        @plsc.parallel_loop(0, _VECS, unroll=16)
        def _(i):
            off = i * _L
            idx = inp_v[pl.ds(off, _L)]
            out_v[pl.ds(off, _L)] = plsc.load_gather(tab_v, [idx])

        pltpu.sync_copy(out_v, out_hbm.at[pl.ds(base, _CHUNK)])

    return lookup


_lookup = _build_sc_call()


@jax.jit
def kernel(inputs, table):
    flat = inputs.reshape(_TOTAL)
    out = _lookup(flat, table)
    return out.reshape(inputs.shape)


# speedup vs baseline: 294.8857x; 1.2702x over previous
import functools

import jax
import jax.numpy as jnp
from jax import lax
from jax.experimental import pallas as pl
from jax.experimental.pallas import tpu as pltpu, tpu_sc as plsc

_NC = 2
_NS = 16
_NW = _NC * _NS
_L = 16

_ROWS = 4096
_COLS = 200
_ROWS_PER_TILE = _ROWS // _NW
_FULL_WIN = _COLS // _L
_TAIL_OFF = _COLS - _L
_TABLE = 256


def _build_sc_call():
    mesh = plsc.VectorSubcoreMesh(core_axis_name="c", subcore_axis_name="s")

    @functools.partial(
        pl.kernel,
        out_type=jax.ShapeDtypeStruct((_ROWS, _COLS), jnp.int32),
        mesh=mesh,
        scratch_types=[
            pltpu.VMEM((_TABLE,), jnp.int32),
            pltpu.VMEM((_ROWS_PER_TILE, _COLS), jnp.int32),
            pltpu.VMEM((_ROWS_PER_TILE, _COLS), jnp.int32),
        ],
        compiler_params=pltpu.CompilerParams(needs_layout_passes=False),
    )
    def lookup(inp_hbm, tab_hbm, out_hbm, tab_v, inp_v, out_v):
        wid = lax.axis_index("s") * _NC + lax.axis_index("c")
        r0 = wid * _ROWS_PER_TILE
        pltpu.sync_copy(tab_hbm, tab_v)
        pltpu.sync_copy(inp_hbm.at[pl.ds(r0, _ROWS_PER_TILE), :], inp_v)

        @plsc.parallel_loop(0, _ROWS_PER_TILE, unroll=2)
        def _(r):
            for w in range(_FULL_WIN):
                off = w * _L
                idx = inp_v[r, pl.ds(off, _L)]
                out_v[r, pl.ds(off, _L)] = plsc.load_gather(tab_v, [idx])
            idx = inp_v[r, pl.ds(_TAIL_OFF, _L)]
            out_v[r, pl.ds(_TAIL_OFF, _L)] = plsc.load_gather(tab_v, [idx])

        pltpu.sync_copy(out_v, out_hbm.at[pl.ds(r0, _ROWS_PER_TILE), :])

    return lookup


_lookup = _build_sc_call()


@jax.jit
def kernel(inputs, table):
    return _lookup(inputs, table)
